# baseline (speedup 1.0000x reference)
import jax
import jax.numpy as jnp
from jax import lax
from jax.experimental import pallas as pl
from jax.experimental.pallas import tpu as pltpu
from jax.experimental.pallas import tpu_sc as plsc

N_USERS = 10000
N_EDGES = 320000
D_EDGE = 16
MEM_DIM = 172
B = 20000
HID = 128
NUM_CLASSES = 2

NC, NS = 2, 16
NW = NC * NS
NP = 10240
BP = 20480
MP = 176
ROW_BLK = 1024
NPW = NP // NW
CHUNK = 2 * BP // NW
PAD_NODE = NP - 1


def _sc_mesh():
    return plsc.VectorSubcoreMesh(core_axis_name="c", subcore_axis_name="s",
                                  num_cores=NC, num_subcores=NS)


def _wid():
    return lax.axis_index("s") * NC + lax.axis_index("c")


def _scan_body(all_nodes_hbm, partial_hbm, chunk_v, pos_v):
    wid = _wid()
    pltpu.sync_copy(all_nodes_hbm.at[pl.ds(wid * CHUNK, CHUNK)], chunk_v)

    def init(i, carry):
        pos_v[pl.ds(i * 16, 16)] = jnp.full((16,), -1, jnp.int32)
        return carry

    lax.fori_loop(0, NP // 16, init, 0)

    iota = lax.iota(jnp.int32, 16)
    jbase = wid * CHUNK

    def scan(c, carry):
        nodes = chunk_v[pl.ds(c * 16, 16)]
        j = jbase + c * 16 + iota
        key = (nodes << 16) | j
        skey = jnp.sort(key)
        nxt = skey.at[jnp.minimum(iota + 1, 15)].get(mode="promise_in_bounds")
        node_s = skey >> 16
        winner = (node_s != (nxt >> 16)) | (iota == 15)
        plsc.store_scatter(pos_v, [node_s], skey & 0xFFFF, mask=winner)
        return carry

    lax.fori_loop(0, CHUNK // 16, scan, 0)
    pltpu.sync_copy(pos_v, partial_hbm.at[wid])


def _scan_stage(all_nodes_p):
    return pl.kernel(
        _scan_body,
        compiler_params=pltpu.CompilerParams(use_tc_tiling_on_sc=False, needs_layout_passes=False),
        out_type=jax.ShapeDtypeStruct((NW, NP), jnp.int32),
        mesh=_sc_mesh(),
        scratch_types=[
            pltpu.VMEM((CHUNK,), jnp.int32),
            pltpu.VMEM((NP,), jnp.int32),
        ],
    )(all_nodes_p)


def _meta_body(partial_hbm, all_nodes_hbm, et_hbm, ei_hbm, lu_hbm,
               other_hbm, eidx_hbm, dt_hbm,
               parts_v, all_v, et_v, ei_v, lu_v, oth_v, eid_v, dt_v):
    wid = _wid()
    base = wid * NPW
    pltpu.sync_copy(all_nodes_hbm, all_v)
    pltpu.sync_copy(et_hbm, et_v)
    pltpu.sync_copy(ei_hbm, ei_v)
    pltpu.sync_copy(lu_hbm.at[pl.ds(base, NPW)], lu_v)
    for k in range(NW):
        pltpu.sync_copy(partial_hbm.at[k, pl.ds(base, NPW)], parts_v.at[k])

    for c in range(NPW // 16):
        sl = pl.ds(c * 16, 16)
        p = parts_v[0, sl]
        for k in range(1, NW):
            p = jnp.maximum(p, parts_v[k, sl])
        p0 = jnp.maximum(p, 0)
        side_dst = p0 >= BP
        e = jnp.where(side_dst, p0 - BP, p0)
        oidx = jnp.where(side_dst, e, e + BP)
        oth_v[sl] = plsc.load_gather(all_v, [oidx])
        eid_v[sl] = plsc.load_gather(ei_v, [e])
        dt_v[sl] = plsc.load_gather(et_v, [e]) - lu_v[sl]

    pltpu.sync_copy(oth_v, other_hbm.at[pl.ds(base, NPW)])
    pltpu.sync_copy(eid_v, eidx_hbm.at[pl.ds(base, NPW)])
    pltpu.sync_copy(dt_v, dt_hbm.at[pl.ds(base, NPW)])


def _meta_stage(partial, all_nodes_p, et_p, ei_p, lu_p):
    return pl.kernel(
        _meta_body,
        compiler_params=pltpu.CompilerParams(use_tc_tiling_on_sc=False, needs_layout_passes=False),
        out_type=(jax.ShapeDtypeStruct((NP,), jnp.int32),
                  jax.ShapeDtypeStruct((NP,), jnp.int32),
                  jax.ShapeDtypeStruct((NP,), jnp.float32)),
        mesh=_sc_mesh(),
        scratch_types=[
            pltpu.VMEM((NW, NPW), jnp.int32),
            pltpu.VMEM((2 * BP,), jnp.int32),
            pltpu.VMEM((BP,), jnp.float32),
            pltpu.VMEM((BP,), jnp.int32),
            pltpu.VMEM((NPW,), jnp.float32),
            pltpu.VMEM((NPW,), jnp.int32),
            pltpu.VMEM((NPW,), jnp.int32),
            pltpu.VMEM((NPW,), jnp.float32),
        ],
    )(partial, all_nodes_p, et_p, ei_p, lu_p)


GC = 80


def _rows_body(mem_hbm, ef_hbm, other_hbm, eidx_hbm, mo_hbm, efg_hbm,
               oi_v, ei_v, mrows_v, erows_v, sem1, sem2):
    wid = _wid()
    base = wid * NPW
    pltpu.sync_copy(other_hbm.at[pl.ds(base, NPW)], oi_v)
    pltpu.sync_copy(eidx_hbm.at[pl.ds(base, NPW)], ei_v)
    copies = []
    for k in range(NPW // GC):
        sl = pl.ds(k * GC, GC)
        copies.append(pltpu.async_copy(mem_hbm.at[oi_v.at[sl]],
                                       mrows_v.at[sl], sem1))
        copies.append(pltpu.async_copy(ef_hbm.at[ei_v.at[sl]],
                                       erows_v.at[sl], sem2))
    for cp in copies:
        cp.wait()
    pltpu.sync_copy(mrows_v, mo_hbm.at[pl.ds(base, NPW)])
    pltpu.sync_copy(erows_v, efg_hbm.at[pl.ds(base, NPW)])


def _rows_stage(mem_p, edge_features, other, eidx_g):
    return pl.kernel(
        _rows_body,
        compiler_params=pltpu.CompilerParams(use_tc_tiling_on_sc=False, needs_layout_passes=False),
        out_type=(jax.ShapeDtypeStruct((NP, MP), jnp.float32),
                  jax.ShapeDtypeStruct((NP, D_EDGE), jnp.float32)),
        mesh=_sc_mesh(),
        scratch_types=[
            pltpu.VMEM((NPW,), jnp.int32),
            pltpu.VMEM((NPW,), jnp.int32),
            pltpu.VMEM((NPW, MP), jnp.float32),
            pltpu.VMEM((NPW, D_EDGE), jnp.float32),
            pltpu.SemaphoreType.DMA,
            pltpu.SemaphoreType.DMA,
        ],
    )(mem_p, edge_features, other, eidx_g)


BW = BP // NW


def _out_gather_body(ln_hbm, src_hbm, out_hbm, idx_v, rows_v, sem):
    wid = _wid()
    base = wid * BW
    pltpu.sync_copy(src_hbm.at[pl.ds(base, BW)], idx_v)
    copies = []
    for k in range(BW // 128):
        sl = pl.ds(k * 128, 128)
        copies.append(pltpu.async_copy(ln_hbm.at[idx_v.at[sl]],
                                       rows_v.at[sl], sem))
    for cp in copies:
        cp.wait()
    pltpu.sync_copy(rows_v, out_hbm.at[pl.ds(base, BW)])


def _out_gather_stage(logits_node, src_p):
    return pl.kernel(
        _out_gather_body,
        compiler_params=pltpu.CompilerParams(use_tc_tiling_on_sc=False, needs_layout_passes=False),
        out_type=jax.ShapeDtypeStruct((BP, 16), jnp.float32),
        mesh=_sc_mesh(),
        scratch_types=[
            pltpu.VMEM((BW,), jnp.int32),
            pltpu.VMEM((BW, 16), jnp.float32),
            pltpu.SemaphoreType.DMA,
        ],
    )(logits_node, src_p)


def _dense_tc_kernel(mem_ref, mo_ref, ef_ref, dt_ref,
                     vr_m, vr_mo, vr_ef, vr_t,
                     vz_m, vz_mo, vz_ef, vz_t,
                     wn_m, wn_mo, wn_ef, wn_t,
                     whn, w1t, w2t,
                     br, bz, bn, bhn, b1, b2, tw, tb,
                     out_ref):
    def mm(a, b):
        return lax.dot_general(a, b, (((1,), (0,)), ((), ())),
                               preferred_element_type=jnp.float32)

    m = mem_ref[...]
    mo = mo_ref[...]
    ef = ef_ref[...]
    tenc = jnp.cos(dt_ref[...] * tw[...] + tb[...])

    ar = mm(m, vr_m[...]) + mm(mo, vr_mo[...]) + mm(ef, vr_ef[...]) \
        + mm(tenc, vr_t[...]) + br[...]
    az = mm(m, vz_m[...]) + mm(mo, vz_mo[...]) + mm(ef, vz_ef[...]) \
        + mm(tenc, vz_t[...]) + bz[...]
    i_n = mm(m, wn_m[...]) + mm(mo, wn_mo[...]) + mm(ef, wn_ef[...]) \
        + mm(tenc, wn_t[...]) + bn[...]
    h_n = mm(m, whn[...]) + bhn[...]

    r = jax.nn.sigmoid(ar)
    z = jax.nn.sigmoid(az)
    n = jnp.tanh(i_n + r * h_n)
    h_new = (1.0 - z) * n + z * m

    h = jnp.maximum(mm(h_new, w1t[...]) + b1[...], 0.0)
    out_ref[...] = mm(h, w2t[...]) + b2[...]


def _dense_stage(mem_p, mo, efg, dt, weights):
    grid = NP // ROW_BLK
    row_bs = lambda c: pl.BlockSpec((ROW_BLK, c), lambda i: (i, 0))
    const_bs = lambda shp: pl.BlockSpec(shp, lambda i: (0, 0))
    in_specs = [row_bs(MP), row_bs(MP), row_bs(D_EDGE), row_bs(1)] + \
        [const_bs(w.shape) for w in weights]
    return pl.pallas_call(
        _dense_tc_kernel,
        grid=(grid,),
        in_specs=in_specs,
        out_specs=pl.BlockSpec((ROW_BLK, 16), lambda i: (i, 0)),
        out_shape=jax.ShapeDtypeStruct((NP, 16), jnp.float32),
    )(mem_p, mo, efg, dt, *weights)


def _prep_weights(W_ih, W_hh, b_ih, b_hh, W1, b1, W2, b2, time_w, time_b):
    D = MEM_DIM

    def padw(w):
        k, n = w.shape
        return jnp.pad(w, ((0, -k % 16), (0, -n % 16)))

    def gate(w_rows):
        wm = w_rows[:, 0:D].T
        wmo = w_rows[:, D:2 * D].T
        wef = w_rows[:, 2 * D:2 * D + D_EDGE].T
        wt = w_rows[:, 2 * D + D_EDGE:].T
        return wm, wmo, wef, wt

    wr_m, wr_mo, wr_ef, wr_t = gate(W_ih[0:D])
    wz_m, wz_mo, wz_ef, wz_t = gate(W_ih[D:2 * D])
    wn_m, wn_mo, wn_ef, wn_t = gate(W_ih[2 * D:3 * D])
    whr = W_hh[0:D].T
    whz = W_hh[D:2 * D].T
    whn = W_hh[2 * D:3 * D].T

    row = lambda v: jnp.pad(v, (0, -v.shape[0] % 16)).reshape(1, -1)
    return (
        padw(wr_m + whr), padw(wr_mo), padw(wr_ef), padw(wr_t),
        padw(wz_m + whz), padw(wz_mo), padw(wz_ef), padw(wz_t),
        padw(wn_m), padw(wn_mo), padw(wn_ef), padw(wn_t),
        padw(whn), padw(W1.T), padw(W2.T),
        row(b_ih[0:D] + b_hh[0:D]),
        row(b_ih[D:2 * D] + b_hh[D:2 * D]),
        row(b_ih[2 * D:3 * D]),
        row(b_hh[2 * D:3 * D]),
        row(b1), row(b2),
        row(time_w), row(time_b),
    )


def kernel(source_nodes, destination_nodes, edge_times, edge_idxs,
           edge_features, memory, last_update, time_w, time_b,
           W_ih, W_hh, b_ih, b_hh, W1, b1, W2, b2):
    src = source_nodes.astype(jnp.int32)
    dst = destination_nodes.astype(jnp.int32)
    eidx = edge_idxs.astype(jnp.int32)

    src_p = jnp.pad(src, (0, BP - B), constant_values=PAD_NODE)
    dst_p = jnp.pad(dst, (0, BP - B), constant_values=PAD_NODE)
    all_nodes_p = jnp.concatenate([src_p, dst_p])
    et_p = jnp.pad(edge_times, (0, BP - B))
    ei_p = jnp.pad(eidx, (0, BP - B))
    mem_p = jnp.pad(memory, ((0, NP - N_USERS), (0, MP - MEM_DIM)))
    lu_p = jnp.pad(last_update, (0, NP - N_USERS))

    partial = _scan_stage(all_nodes_p)
    other, eidx_g, dt = _meta_stage(partial, all_nodes_p, et_p, ei_p, lu_p)
    mo, efg = _rows_stage(mem_p, edge_features, other, eidx_g)

    weights = _prep_weights(W_ih, W_hh, b_ih, b_hh, W1, b1, W2, b2,
                            time_w, time_b)
    logits_node = _dense_stage(mem_p, mo, efg, dt[:, None], weights)

    src_g = jnp.pad(src, (0, BP - B))
    logits = _out_gather_stage(logits_node, src_g)

    return logits[:B, :NUM_CLASSES]


# speedup vs baseline: 1.5147x; 1.5147x over previous
import jax
import jax.numpy as jnp
from jax import lax
from jax.experimental import pallas as pl
from jax.experimental.pallas import tpu as pltpu
from jax.experimental.pallas import tpu_sc as plsc

N_USERS = 10000
N_EDGES = 320000
D_EDGE = 16
MEM_DIM = 172
B = 20000
HID = 128
NUM_CLASSES = 2

NC, NS = 2, 16
NW = NC * NS
NP = 10240
BP = 20480
MP = 176
ROW_BLK = 1024
NPW = NP // NW
CHUNK = 2 * BP // NW
PAD_NODE = NP - 1


def _sc_mesh():
    return plsc.VectorSubcoreMesh(core_axis_name="c", subcore_axis_name="s",
                                  num_cores=NC, num_subcores=NS)


def _wid():
    return lax.axis_index("s") * NC + lax.axis_index("c")


def _scan_body(all_nodes_hbm, partial_hbm, chunk_v, pos_v):
    wid = _wid()
    pltpu.sync_copy(all_nodes_hbm.at[pl.ds(wid * CHUNK, CHUNK)], chunk_v)

    def init(i, carry):
        pos_v[pl.ds(i * 16, 16)] = jnp.full((16,), -1, jnp.int32)
        return carry

    lax.fori_loop(0, NP // 16, init, 0)

    iota = lax.iota(jnp.int32, 16)
    jbase = wid * CHUNK

    def scan(c, carry):
        nodes = chunk_v[pl.ds(c * 16, 16)]
        j = jbase + c * 16 + iota
        key = (nodes << 16) | j
        skey = jnp.sort(key)
        nxt = skey.at[jnp.minimum(iota + 1, 15)].get(mode="promise_in_bounds")
        node_s = skey >> 16
        winner = (node_s != (nxt >> 16)) | (iota == 15)
        plsc.store_scatter(pos_v, [node_s], skey & 0xFFFF, mask=winner)
        return carry

    lax.fori_loop(0, CHUNK // 16, scan, 0)
    pltpu.sync_copy(pos_v, partial_hbm.at[wid])


def _scan_stage(all_nodes_p):
    return pl.kernel(
        _scan_body,
        compiler_params=pltpu.CompilerParams(use_tc_tiling_on_sc=False, needs_layout_passes=False),
        out_type=jax.ShapeDtypeStruct((NW, NP), jnp.int32),
        mesh=_sc_mesh(),
        scratch_types=[
            pltpu.VMEM((CHUNK,), jnp.int32),
            pltpu.VMEM((NP,), jnp.int32),
        ],
    )(all_nodes_p)


GC = 80


_CHUNKS = ((0, 128), (128, 128), (256, 64))


def _chunked_gather(tbl_hbm, idx_v, out_hbm, base, b0, b1, sem):
    bufs = (b0, b1)
    cps = {}
    for i, (st, sz) in enumerate(_CHUNKS):
        if i >= 2:
            pst, psz = _CHUNKS[i - 2]
            cps[i - 2].wait()
            pltpu.sync_copy(bufs[(i - 2) % 2].at[pl.ds(0, psz)],
                            out_hbm.at[pl.ds(base + pst, psz)])
        cps[i] = pltpu.async_copy(tbl_hbm.at[idx_v.at[pl.ds(st, sz)]],
                                  bufs[i % 2].at[pl.ds(0, sz)], sem)
    for i in (len(_CHUNKS) - 2, len(_CHUNKS) - 1):
        st, sz = _CHUNKS[i]
        cps[i].wait()
        pltpu.sync_copy(bufs[i % 2].at[pl.ds(0, sz)],
                        out_hbm.at[pl.ds(base + st, sz)])


def _gather_body(partial_hbm, mt_hbm, lu_hbm, mema_hbm, memb_hbm, efph_hbm,
                 moa_hbm, mob_hbm, efg_hbm, dt_hbm,
                 parts_v, e_v, side_v, oth_v, eidx2_v, lu_v, dt_v,
                 meta_v, b0_v, b1_v, efg_v, sem_a, sem_b):
    wid = _wid()
    base = wid * NPW
    lu_cp = pltpu.async_copy(lu_hbm.at[pl.ds(base, NPW)], lu_v, sem_b)
    pltpu.sync_copy(partial_hbm.at[:, pl.ds(base, NPW)], parts_v)

    for c in range(NPW // 16):
        sl = pl.ds(c * 16, 16)
        p = parts_v[0, sl]
        for k in range(1, NW):
            p = jnp.maximum(p, parts_v[k, sl])
        p0 = jnp.maximum(p, 0)
        side_dst = p0 >= BP
        e_v[sl] = jnp.where(side_dst, p0 - BP, p0)
        side_v[sl] = jnp.where(side_dst, 1, 0)

    meta_cps = [pltpu.async_copy(mt_hbm.at[e_v.at[pl.ds(st, sz)]],
                                 meta_v.at[pl.ds(st, sz)], sem_a)
                for st, sz in _CHUNKS]
    for cp in meta_cps:
        cp.wait()
    lu_cp.wait()

    iota = lax.iota(jnp.int32, 16)
    zero = jnp.zeros((16,), jnp.int32)
    for c in range(NPW // 16):
        sl = pl.ds(c * 16, 16)
        rows = c * 16 + iota
        src = plsc.load_gather(meta_v, [rows, zero])
        dst = plsc.load_gather(meta_v, [rows, zero + 1])
        ei = plsc.load_gather(meta_v, [rows, zero + 2])
        etb = plsc.load_gather(meta_v, [rows, zero + 3])
        sideb = side_v[sl] > 0
        oth_v[sl] = jnp.where(sideb, src, dst)
        dt_v[sl] = plsc.bitcast(etb, jnp.float32) - lu_v[sl]
        ebase = ((ei >> 7) << 10) + (ei & 127)
        for f in range(D_EDGE):
            off = (f // 8) * (N_EDGES * 8) + (f % 8) * 128
            plsc.store_scatter(eidx2_v, [rows * D_EDGE + f], ebase + off)

    pltpu.sync_copy(dt_v, dt_hbm.at[pl.ds(base, NPW)])

    ef_cps = [pltpu.async_copy(efph_hbm.at[eidx2_v.at[pl.ds(j * 128, 128)]],
                               efg_v.at[pl.ds(j * 128, 128)], sem_b)
              for j in range(NPW * D_EDGE // 128)]

    _chunked_gather(mema_hbm, oth_v, moa_hbm, base, b0_v, b1_v, sem_a)
    _chunked_gather(memb_hbm, oth_v, mob_hbm, base, b0_v, b1_v, sem_a)

    for cp in ef_cps:
        cp.wait()
    pltpu.sync_copy(efg_v, efg_hbm.at[pl.ds(base * D_EDGE, NPW * D_EDGE)])


def _gather_stage(partial, meta_tbl, lu_p, mem_ta, mem_tb, ef_phys):
    return pl.kernel(
        _gather_body,
        compiler_params=pltpu.CompilerParams(use_tc_tiling_on_sc=False, needs_layout_passes=False),
        out_type=(jax.ShapeDtypeStruct((NP, 128), jnp.float32),
                  jax.ShapeDtypeStruct((NP, 128), jnp.float32),
                  jax.ShapeDtypeStruct((NP * D_EDGE,), jnp.float32),
                  jax.ShapeDtypeStruct((NP,), jnp.float32)),
        mesh=_sc_mesh(),
        scratch_types=[
            pltpu.VMEM((NW, NPW), jnp.int32),
            pltpu.VMEM((NPW,), jnp.int32),
            pltpu.VMEM((NPW,), jnp.int32),
            pltpu.VMEM((NPW,), jnp.int32),
            pltpu.VMEM((NPW * D_EDGE,), jnp.int32),
            pltpu.VMEM((NPW,), jnp.float32),
            pltpu.VMEM((NPW,), jnp.float32),
            pltpu.VMEM((NPW, 16), jnp.int32),
            pltpu.VMEM((128, 128), jnp.float32),
            pltpu.VMEM((128, 128), jnp.float32),
            pltpu.VMEM((NPW * D_EDGE,), jnp.float32),
            pltpu.SemaphoreType.DMA,
            pltpu.SemaphoreType.DMA,
        ],
    )(partial, meta_tbl, lu_p, mem_ta, mem_tb, ef_phys)


BW = BP // NW


def _out_gather_body(ln_hbm, src_hbm, out_hbm, idx_v, rows_v, sem):
    wid = _wid()
    base = wid * BW
    pltpu.sync_copy(src_hbm.at[pl.ds(base, BW)], idx_v)
    copies = []
    for k in range(BW // 128):
        sl = pl.ds(k * 128, 128)
        copies.append(pltpu.async_copy(ln_hbm.at[idx_v.at[sl]],
                                       rows_v.at[sl], sem))
    for cp in copies:
        cp.wait()
    pltpu.sync_copy(rows_v, out_hbm.at[pl.ds(base, BW)])


def _out_gather_stage(logits_node, src_p):
    return pl.kernel(
        _out_gather_body,
        compiler_params=pltpu.CompilerParams(use_tc_tiling_on_sc=False, needs_layout_passes=False),
        out_type=jax.ShapeDtypeStruct((BP, 16), jnp.float32),
        mesh=_sc_mesh(),
        scratch_types=[
            pltpu.VMEM((BW,), jnp.int32),
            pltpu.VMEM((BW, 16), jnp.float32),
            pltpu.SemaphoreType.DMA,
        ],
    )(logits_node, src_p)


def _dense_tc_kernel(ma_ref, mb_ref, moa_ref, mob_ref, ef_ref, dt_ref,
                     vr_m, vr_mo, vr_ef, vr_t,
                     vz_m, vz_mo, vz_ef, vz_t,
                     wn_m, wn_mo, wn_ef, wn_t,
                     whn, w1t, w2t,
                     br, bz, bn, bhn, b1, b2, tw, tb,
                     out_ref):
    def mm(a, b):
        return lax.dot_general(a, b, (((1,), (0,)), ((), ())),
                               preferred_element_type=jnp.float32)

    def mm2(a, b, w):
        return mm(a, w[0:128, :]) + mm(b, w[128:256, :])

    ma = ma_ref[...]
    mb = mb_ref[...]
    moa = moa_ref[...]
    mob = mob_ref[...]
    ef = ef_ref[...]
    tenc = jnp.cos(dt_ref[...] * tw[...] + tb[...])

    ar = mm2(ma, mb, vr_m[...]) + mm2(moa, mob, vr_mo[...]) \
        + mm(ef, vr_ef[...]) + mm(tenc, vr_t[...]) + br[...]
    az = mm2(ma, mb, vz_m[...]) + mm2(moa, mob, vz_mo[...]) \
        + mm(ef, vz_ef[...]) + mm(tenc, vz_t[...]) + bz[...]
    i_n = mm2(ma, mb, wn_m[...]) + mm2(moa, mob, wn_mo[...]) \
        + mm(ef, wn_ef[...]) + mm(tenc, wn_t[...]) + bn[...]
    h_n = mm2(ma, mb, whn[...]) + bhn[...]

    r = jax.nn.sigmoid(ar)
    z = jax.nn.sigmoid(az)
    n = jnp.tanh(i_n + r * h_n)
    za = z[:, 0:128]
    zb = z[:, 128:176]
    ha = (1.0 - za) * n[:, 0:128] + za * ma
    hb = (1.0 - zb) * n[:, 128:176] + zb * mb[:, 0:48]

    h = jnp.maximum(mm(ha, w1t[0:128, :]) + mm(hb, w1t[128:176, :])
                    + b1[...], 0.0)
    out_ref[...] = mm(h, w2t[...]) + b2[...]


def _dense_stage(mem_ta, mem_tb, moa, mob, efg, dt, weights):
    grid = NP // ROW_BLK
    row_bs = lambda c: pl.BlockSpec((ROW_BLK, c), lambda i: (i, 0))
    const_bs = lambda shp: pl.BlockSpec(shp, lambda i: (0, 0))
    in_specs = [row_bs(128), row_bs(128), row_bs(128), row_bs(128),
                row_bs(D_EDGE), row_bs(1)] + \
        [const_bs(w.shape) for w in weights]
    return pl.pallas_call(
        _dense_tc_kernel,
        grid=(grid,),
        in_specs=in_specs,
        out_specs=pl.BlockSpec((ROW_BLK, 16), lambda i: (i, 0)),
        out_shape=jax.ShapeDtypeStruct((NP, 16), jnp.float32),
    )(mem_ta, mem_tb, moa, mob, efg, dt, *weights)


def _prep_weights(W_ih, W_hh, b_ih, b_hh, W1, b1, W2, b2, time_w, time_b):
    D = MEM_DIM

    def padw(w):
        k, n = w.shape
        return jnp.pad(w, ((0, -k % 16), (0, -n % 16)))

    def padw256(w):
        k, n = w.shape
        return jnp.pad(w, ((0, 256 - k), (0, -n % 16)))

    def gate(w_rows):
        wm = w_rows[:, 0:D].T
        wmo = w_rows[:, D:2 * D].T
        wef = w_rows[:, 2 * D:2 * D + D_EDGE].T
        wt = w_rows[:, 2 * D + D_EDGE:].T
        return wm, wmo, wef, wt

    wr_m, wr_mo, wr_ef, wr_t = gate(W_ih[0:D])
    wz_m, wz_mo, wz_ef, wz_t = gate(W_ih[D:2 * D])
    wn_m, wn_mo, wn_ef, wn_t = gate(W_ih[2 * D:3 * D])
    whr = W_hh[0:D].T
    whz = W_hh[D:2 * D].T
    whn = W_hh[2 * D:3 * D].T

    row = lambda v: jnp.pad(v, (0, -v.shape[0] % 16)).reshape(1, -1)
    return (
        padw256(wr_m + whr), padw256(wr_mo), padw(wr_ef), padw(wr_t),
        padw256(wz_m + whz), padw256(wz_mo), padw(wz_ef), padw(wz_t),
        padw256(wn_m), padw256(wn_mo), padw(wn_ef), padw(wn_t),
        padw256(whn), padw(W1.T), padw(W2.T),
        row(b_ih[0:D] + b_hh[0:D]),
        row(b_ih[D:2 * D] + b_hh[D:2 * D]),
        row(b_ih[2 * D:3 * D]),
        row(b_hh[2 * D:3 * D]),
        row(b1), row(b2),
        row(time_w), row(time_b),
    )


def kernel(source_nodes, destination_nodes, edge_times, edge_idxs,
           edge_features, memory, last_update, time_w, time_b,
           W_ih, W_hh, b_ih, b_hh, W1, b1, W2, b2):
    src = source_nodes.astype(jnp.int32)
    dst = destination_nodes.astype(jnp.int32)
    eidx = edge_idxs.astype(jnp.int32)

    src_p = jnp.pad(src, (0, BP - B), constant_values=PAD_NODE)
    dst_p = jnp.pad(dst, (0, BP - B), constant_values=PAD_NODE)
    all_nodes_p = jnp.concatenate([src_p, dst_p])
    et_p = jnp.pad(edge_times, (0, BP - B))
    ei_p = jnp.pad(eidx, (0, BP - B))
    lu_p = jnp.pad(last_update, (0, NP - N_USERS))

    etb = lax.bitcast_convert_type(et_p, jnp.int32)
    meta_tbl = jnp.pad(jnp.stack([src_p, dst_p, ei_p, etb], axis=1),
                       ((0, 0), (0, 12)))

    mem_ta = jnp.pad(memory[:, 0:128], ((0, NP - N_USERS), (0, 0)))
    mem_tb = jnp.pad(memory[:, 128:MEM_DIM],
                     ((0, NP - N_USERS), (0, 256 - MEM_DIM)))
    ef_phys = jnp.ravel(jnp.transpose(
        edge_features.reshape(N_EDGES // 128, 128, 2, 8), (2, 0, 3, 1)))

    partial = _scan_stage(all_nodes_p)
    moa, mob, efg_flat, dt = _gather_stage(partial, meta_tbl, lu_p,
                                           mem_ta, mem_tb, ef_phys)
    efg = efg_flat.reshape(NP, D_EDGE)

    weights = _prep_weights(W_ih, W_hh, b_ih, b_hh, W1, b1, W2, b2,
                            time_w, time_b)
    logits_node = _dense_stage(mem_ta, mem_tb, moa, mob, efg,
                               dt[:, None], weights)

    src_g = jnp.pad(src, (0, BP - B))
    logits = _out_gather_stage(logits_node, src_g)

    return logits[:B, :NUM_CLASSES]
